# Initial kernel scaffold; baseline (speedup 1.0000x reference)
import jax
import jax.numpy as jnp
from jax.experimental import pallas as pl


def kernel(quantum_embeddings):
    raise NotImplementedError("write your pallas kernel here")


# speedup vs baseline: 34.2662x; 34.2662x over previous
import jax
import jax.numpy as jnp
from jax.experimental import pallas as pl

_PERSISTENCE_THRESHOLD = 0.1
_TAU = 1.0
_K = 5
_ROW_TILE = 256


def _topo_filter_kernel(x_tile_ref, x_full_ref, out_ref):
    x = x_tile_ref[:]
    xa = x_full_ref[:]
    r = x.shape[0]
    n = xa.shape[0]
    sq_r = jnp.sum(x * x, axis=1, keepdims=True)
    sq_c = jnp.sum(xa * xa, axis=1).reshape(1, n)
    gram = jax.lax.dot_general(
        x, xa, (((1,), (1,)), ((), ())),
        preferred_element_type=jnp.float32)
    d2 = sq_r + sq_c - 2.0 * gram
    d = jnp.sqrt(jnp.maximum(d2, 0.0))

    ids = jax.lax.broadcasted_iota(jnp.int32, (r, n), 1)
    work = d
    total = jnp.zeros((r, 1), jnp.float32)
    m0 = None
    for j in range(_K + 1):
        m = jnp.min(work, axis=1, keepdims=True)
        if j == 0:
            m0 = m
        total = total + m
        if j < _K:
            idx = jnp.min(jnp.where(work == m, ids, n), axis=1, keepdims=True)
            work = jnp.where(ids == idx, jnp.float32(jnp.inf), work)

    p = (total - m0) * (1.0 / _K)
    w = jnp.exp(-p / _TAU)
    w = jnp.where(p > _PERSISTENCE_THRESHOLD, w, jnp.float32(0.0))
    out_ref[:] = x * w


def kernel(quantum_embeddings):
    b, s, d = quantum_embeddings.shape
    flat = quantum_embeddings.reshape(-1, d)
    n = flat.shape[0]
    out = pl.pallas_call(
        _topo_filter_kernel,
        grid=(n // _ROW_TILE,),
        in_specs=[
            pl.BlockSpec((_ROW_TILE, d), lambda i: (i, 0)),
            pl.BlockSpec((n, d), lambda i: (0, 0)),
        ],
        out_specs=pl.BlockSpec((_ROW_TILE, d), lambda i: (i, 0)),
        out_shape=jax.ShapeDtypeStruct((n, d), flat.dtype),
    )(flat, flat)
    return out.reshape(b, s, d)


# speedup vs baseline: 72.2597x; 2.1088x over previous
import jax
import jax.numpy as jnp
from jax.experimental import pallas as pl

_PERSISTENCE_THRESHOLD = 0.1
_TAU = 1.0
_K = 5
_ROW_TILE = 256
_LANES = 128


def _topo_filter_kernel(x_tile_ref, x_full_ref, out_ref):
    x = x_tile_ref[:]
    xa = x_full_ref[:]
    r = x.shape[0]
    n = xa.shape[0]
    sq_r = jnp.sum(x * x, axis=1, keepdims=True)
    sq_c = jnp.sum(xa * xa, axis=1).reshape(1, n)
    gram = jax.lax.dot_general(
        x, xa, (((1,), (1,)), ((), ())),
        preferred_element_type=jnp.float32)
    d2 = jnp.maximum(sq_r + sq_c - 2.0 * gram, 0.0)

    inf = jnp.float32(jnp.inf)
    slots = [jnp.full((r, _LANES), inf, jnp.float32) for _ in range(_K + 1)]
    for c in range(n // _LANES):
        t = jax.lax.slice(d2, (0, c * _LANES), (r, (c + 1) * _LANES))
        for s in range(_K + 1):
            lo = jnp.minimum(slots[s], t)
            t = jnp.maximum(slots[s], t)
            slots[s] = lo

    cand = jnp.concatenate(slots, axis=1)

    need = jnp.full((r, 1), jnp.float32(_K + 1))
    total = jnp.zeros((r, 1), jnp.float32)
    m0 = None
    for j in range(_K + 1):
        m = jnp.min(cand, axis=1, keepdims=True)
        dist = jnp.sqrt(m)
        if j == 0:
            m0 = dist
        eq = cand == m
        cnt = jnp.sum(eq.astype(jnp.float32), axis=1, keepdims=True)
        take = jnp.minimum(cnt, need)
        need = need - take
        total = total + dist * take
        if j < _K:
            cand = jnp.where(eq, inf, cand)

    p = (total - m0) * (1.0 / _K)
    w = jnp.exp(-p / _TAU)
    w = jnp.where(p > _PERSISTENCE_THRESHOLD, w, jnp.float32(0.0))
    out_ref[:] = x * w


def kernel(quantum_embeddings):
    b, s, d = quantum_embeddings.shape
    flat = quantum_embeddings.reshape(-1, d)
    n = flat.shape[0]
    out = pl.pallas_call(
        _topo_filter_kernel,
        grid=(n // _ROW_TILE,),
        in_specs=[
            pl.BlockSpec((_ROW_TILE, d), lambda i: (i, 0)),
            pl.BlockSpec((n, d), lambda i: (0, 0)),
        ],
        out_specs=pl.BlockSpec((_ROW_TILE, d), lambda i: (i, 0)),
        out_shape=jax.ShapeDtypeStruct((n, d), flat.dtype),
    )(flat, flat)
    return out.reshape(b, s, d)
